# baseline (speedup 1.0000x reference)
import jax
import jax.numpy as jnp
from jax import lax
from jax.experimental import pallas as pl
from jax.experimental.pallas import tpu as pltpu
from jax.experimental.pallas import tpu_sc as plsc

D_MODEL = 1024
N_ROWS = 8192
N_WORKERS = 32
ROWS_PER_WORKER = N_ROWS // N_WORKERS
CHUNK = 16
N_CHUNKS = ROWS_PER_WORKER // CHUNK
NXB = 4
NPB = 3
LEAD = 2
VECS_PER_ROW = D_MODEL // 16

_mesh = plsc.VectorSubcoreMesh(core_axis_name="c", subcore_axis_name="s")

_scratch = (
    [pltpu.VMEM((ROWS_PER_WORKER,), jnp.int32)]
    + [pltpu.VMEM((CHUNK, D_MODEL), jnp.float32) for _ in range(NXB + NPB)]
    + [pltpu.SemaphoreType.DMA for _ in range(2 * NXB + NPB)]
)


@pl.kernel(
    mesh=_mesh,
    out_type=jax.ShapeDtypeStruct((N_ROWS, D_MODEL), jnp.float32),
    scratch_types=_scratch,
)
def _lookup_add(x_hbm, ids_hbm, pe_hbm, out_hbm, idx_all, *bufs):
    x_b = list(bufs[:NXB])
    pe_b = list(bufs[NXB:NXB + NPB])
    sems = bufs[NXB + NPB:]
    sx = list(sems[:NXB])
    so = list(sems[NXB:2 * NXB])
    sp = list(sems[2 * NXB:])

    wid = lax.axis_index("s") * 2 + lax.axis_index("c")
    base = wid * ROWS_PER_WORKER

    def start_xload(ch):
        row0 = base + ch * CHUNK
        return pltpu.async_copy(
            x_hbm.at[pl.ds(row0, CHUNK), :], x_b[ch % NXB], sx[ch % NXB])

    def start_gather(ch):
        return pltpu.async_copy(
            pe_hbm.at[idx_all.at[pl.ds(ch * CHUNK, CHUNK)]],
            pe_b[ch % NPB], sp[ch % NPB])

    def start_load(ch):
        return start_xload(ch), start_gather(ch)

    first_x = [start_xload(ch) for ch in range(LEAD + 1)]
    pltpu.sync_copy(ids_hbm.at[pl.ds(base, ROWS_PER_WORKER)], idx_all)

    loads = {}
    writes = {}
    for ch in range(LEAD):
        loads[ch] = (first_x[ch], start_gather(ch))
    loads[LEAD] = (first_x[LEAD], start_gather(LEAD))

    for ch in range(N_CHUNKS):
        hx, hp = loads.pop(ch)
        hx.wait()
        hp.wait()

        xb, pb = x_b[ch % NXB], pe_b[ch % NPB]

        @plsc.parallel_loop(0, CHUNK * VECS_PER_ROW, unroll=8)
        def _add(t):
            r = t // VECS_PER_ROW
            j = t % VECS_PER_ROW
            sl = pl.ds(j * 16, 16)
            plsc.addupdate(xb.at[r, sl], pb[r, sl])

        row0 = base + ch * CHUNK
        writes[ch] = pltpu.async_copy(
            xb, out_hbm.at[pl.ds(row0, CHUNK), :], so[ch % NXB])
        nxt = ch + LEAD + 1
        if nxt < N_CHUNKS:
            if nxt >= NXB:
                writes.pop(nxt - NXB).wait()
            loads[nxt] = start_load(nxt)

    for ch in sorted(writes):
        writes.pop(ch).wait()


def kernel(x, position_ids, pe_weight):
    B, L, D = x.shape
    xf = x.reshape(B * L, D)
    ids = position_ids.reshape(B * L).astype(jnp.int32)
    out = _lookup_add(xf, ids, pe_weight)
    return out.reshape(B, L, D)


# speedup vs baseline: 1.0080x; 1.0080x over previous
import jax
import jax.numpy as jnp
from jax import lax
from jax.experimental import pallas as pl
from jax.experimental.pallas import tpu as pltpu
from jax.experimental.pallas import tpu_sc as plsc

D_MODEL = 1024
N_ROWS = 8192
N_WORKERS = 32
ROWS_PER_WORKER = N_ROWS // N_WORKERS
CHUNK = 16
N_CHUNKS = ROWS_PER_WORKER // CHUNK
NXB = 4
NPB = 3
LEAD = 2
VECS_PER_ROW = D_MODEL // 16

_mesh = plsc.VectorSubcoreMesh(core_axis_name="c", subcore_axis_name="s")

_scratch = (
    [pltpu.VMEM((ROWS_PER_WORKER,), jnp.int32)]
    + [pltpu.VMEM((CHUNK, D_MODEL), jnp.float32) for _ in range(NXB + NPB)]
    + [pltpu.SemaphoreType.DMA for _ in range(2 * NXB + NPB)]
)


@pl.kernel(
    mesh=_mesh,
    out_type=jax.ShapeDtypeStruct((N_ROWS, D_MODEL), jnp.float32),
    scratch_types=_scratch,
)
def _lookup_add(x_hbm, ids_hbm, pe_hbm, out_hbm, idx_all, *bufs):
    x_b = list(bufs[:NXB])
    pe_b = list(bufs[NXB:NXB + NPB])
    sems = bufs[NXB + NPB:]
    sx = list(sems[:NXB])
    so = list(sems[NXB:2 * NXB])
    sp = list(sems[2 * NXB:])

    wid = lax.axis_index("s") * 2 + lax.axis_index("c")
    base = wid * ROWS_PER_WORKER

    def start_xload(ch):
        row0 = base + ch * CHUNK
        return pltpu.async_copy(
            x_hbm.at[pl.ds(row0, CHUNK), :], x_b[ch % NXB], sx[ch % NXB])

    def start_gather(ch):
        return pltpu.async_copy(
            pe_hbm.at[idx_all.at[pl.ds(ch * CHUNK, CHUNK)]],
            pe_b[ch % NPB], sp[ch % NPB])

    def start_load(ch):
        return start_xload(ch), start_gather(ch)

    first_x = [start_xload(ch) for ch in range(LEAD)]
    pltpu.sync_copy(ids_hbm.at[pl.ds(base, ROWS_PER_WORKER)], idx_all)

    loads = {}
    writes = {}
    for ch in range(LEAD):
        loads[ch] = (first_x[ch], start_gather(ch))

    for ch in range(N_CHUNKS):
        hx, hp = loads.pop(ch)
        hx.wait()
        hp.wait()

        xb, pb = x_b[ch % NXB], pe_b[ch % NPB]

        @plsc.parallel_loop(0, CHUNK * VECS_PER_ROW, unroll=8)
        def _add(t):
            r = t // VECS_PER_ROW
            j = t % VECS_PER_ROW
            sl = pl.ds(j * 16, 16)
            plsc.addupdate(xb.at[r, sl], pb[r, sl])

        row0 = base + ch * CHUNK
        writes[ch] = pltpu.async_copy(
            xb, out_hbm.at[pl.ds(row0, CHUNK), :], so[ch % NXB])
        nxt = ch + LEAD
        if nxt < N_CHUNKS:
            if nxt >= NXB:
                writes.pop(nxt - NXB).wait()
            loads[nxt] = start_load(nxt)

    for ch in sorted(writes):
        writes.pop(ch).wait()


def kernel(x, position_ids, pe_weight):
    B, L, D = x.shape
    xf = x.reshape(B * L, D)
    ids = position_ids.reshape(B * L).astype(jnp.int32)
    out = _lookup_add(xf, ids, pe_weight)
    return out.reshape(B, L, D)
